# baseline (speedup 1.0000x reference)
import jax
import jax.numpy as jnp
from jax import lax
from jax.experimental import pallas as pl
from jax.experimental.pallas import tpu as pltpu
from jax.experimental.pallas import tpu_sc as plsc

MARGIN = 2.0
K_MARGIN = 0.02
N = 16384
D_FEAT = 64
G = 256
TW = 16

C_POS = 512
NC_POS = N // C_POS

BI2 = 512
CJB = 256
NB2 = N // BI2
NP = N + CJB

SC_NC = 2
SC_NS = 16
SC_NW = SC_NC * SC_NS
SC_CH = N // SC_NW
SC_JB = 128


def _mm(x, y):
    return lax.dot_general(
        x, y, (((1,), (0,)), ((), ())), preferred_element_type=jnp.float32
    )


def _mm_hi(x, y):
    return lax.dot_general(
        x, y, (((1,), (0,)), ((), ())),
        preferred_element_type=jnp.float32,
        precision=lax.Precision.HIGHEST,
    )


def _dot_body(zr_ref, zv_ref, v_ref, g_ref, d_ref, tab_ref):
    d = 1.0 - jnp.sum(zr_ref[...] * zv_ref[...], axis=1, keepdims=True)
    d_ref[...] = d
    vf = v_ref[...].astype(jnp.float32)
    gf = g_ref[...].astype(jnp.float32)
    blk = d.shape[0]
    pad = jnp.zeros((blk, TW - 3), jnp.float32)
    tab_ref[...] = jnp.concatenate([vf, gf, d, pad], axis=1)


def _dot_call(z_r, z_v, v_col, g_col):
    blk = 1024
    return pl.pallas_call(
        _dot_body,
        grid=(N // blk,),
        in_specs=[
            pl.BlockSpec((blk, D_FEAT), lambda b: (b, 0)),
            pl.BlockSpec((blk, D_FEAT), lambda b: (b, 0)),
            pl.BlockSpec((blk, 1), lambda b: (b, 0)),
            pl.BlockSpec((blk, 1), lambda b: (b, 0)),
        ],
        out_specs=[
            pl.BlockSpec((blk, 1), lambda b: (b, 0)),
            pl.BlockSpec((blk, TW), lambda b: (b, 0)),
        ],
        out_shape=[
            jax.ShapeDtypeStruct((N, 1), jnp.float32),
            jax.ShapeDtypeStruct((N, TW), jnp.float32),
        ],
    )(z_r, z_v, v_col, g_col)


def _pos_body(gcol, vcol, grow, vrow, pos_out, ge_out, rc_out, htT, hcT):
    iota_row = lax.broadcasted_iota(jnp.int32, (1, G), 1)
    iota_col = lax.broadcasted_iota(jnp.int32, (G, 1), 0)

    def mats(c):
        gc = gcol[pl.ds(c * C_POS, C_POS), :]
        vc = vcol[pl.ds(c * C_POS, C_POS), :]
        gr = grow[:, pl.ds(c * C_POS, C_POS)]
        vr = vrow[:, pl.ds(c * C_POS, C_POS)]
        a = (gc == iota_row).astype(jnp.float32)
        b = (vc == iota_row).astype(jnp.float32)
        bT = (iota_col == vr).astype(jnp.float32)
        return gc, vc, gr, vr, a, b, bT

    htT[...] = jnp.zeros((G, G), jnp.float32)

    def l1(c, _):
        _, _, _, _, a, _, bT = mats(c)
        htT[...] += _mm(bT, a)
        return 0

    lax.fori_loop(0, NC_POS, l1, 0)

    ht = htT[...]
    slv = (iota_col > iota_row).astype(jnp.float32)
    sug = (iota_col < iota_row).astype(jnp.float32)
    rowcumT = _mm_hi(slv, ht)
    t_row = jnp.sum(ht, axis=0, keepdims=True)
    texT = _mm_hi(t_row, sug)
    offT = texT + rowcumT
    ge_out[...] = (texT + t_row).astype(jnp.int32)
    sumsq = jnp.sum(ht * ht, axis=0, keepdims=True)
    rc_out[...] = (t_row * t_row - sumsq) * 0.5

    hcT[...] = jnp.zeros((G, G), jnp.float32)
    iota_i = lax.broadcasted_iota(jnp.int32, (C_POS, 1), 0)
    iota_j = lax.broadcasted_iota(jnp.int32, (1, C_POS), 1)

    def l2(c, _):
        gc, vc, gr, vr, a, b, bT = mats(c)
        tcT = offT + hcT[...]
        tc_hi = jnp.floor(tcT * (1.0 / 128.0))
        tc_lo = tcT - tc_hi * 128.0
        m = _mm(b, tc_hi) * 128.0 + _mm(b, tc_lo)
        lookup = jnp.sum(a * m, axis=1, keepdims=True)
        meq = (gc == gr) & (vc == vr) & (iota_j < iota_i)
        r = jnp.sum(meq.astype(jnp.float32), axis=1, keepdims=True)
        pos_out[pl.ds(c * C_POS, C_POS), :] = (lookup + r).astype(jnp.int32)
        hcT[...] += _mm(bT, a)
        return 0

    lax.fori_loop(0, NC_POS, l2, 0)


def _pos_call(g_col, v_col, g_row, v_row):
    return pl.pallas_call(
        _pos_body,
        in_specs=[
            pl.BlockSpec((N, 1), lambda: (0, 0)),
            pl.BlockSpec((N, 1), lambda: (0, 0)),
            pl.BlockSpec((1, N), lambda: (0, 0)),
            pl.BlockSpec((1, N), lambda: (0, 0)),
        ],
        out_specs=[
            pl.BlockSpec((N, 1), lambda: (0, 0)),
            pl.BlockSpec((1, G), lambda: (0, 0)),
            pl.BlockSpec((1, G), lambda: (0, 0)),
        ],
        out_shape=[
            jax.ShapeDtypeStruct((N, 1), jnp.int32),
            jax.ShapeDtypeStruct((1, G), jnp.int32),
            jax.ShapeDtypeStruct((1, G), jnp.float32),
        ],
        scratch_shapes=[
            pltpu.VMEM((G, G), jnp.float32),
            pltpu.VMEM((G, G), jnp.float32),
        ],
    )(g_col, v_col, g_row, v_row)


def _sc_scatter_body(tab_hbm, pos_hbm, out_hbm, tab_v, pos_v):
    wid = lax.axis_index("s") * SC_NC + lax.axis_index("c")
    base = wid * SC_CH
    pltpu.sync_copy(tab_hbm.at[pl.ds(base, SC_CH)], tab_v)
    pltpu.sync_copy(pos_hbm.at[wid], pos_v)
    for j in range(SC_CH // SC_JB):
        pltpu.sync_copy(
            tab_v.at[pl.ds(j * SC_JB, SC_JB)], out_hbm.at[pos_v.at[j]]
        )


def _sc_scatter(tab, pos3):
    k = pl.kernel(
        _sc_scatter_body,
        out_type=jax.ShapeDtypeStruct((N, TW), jnp.float32),
        mesh=plsc.VectorSubcoreMesh(core_axis_name="c", subcore_axis_name="s"),
        scratch_types=[
            pltpu.VMEM((SC_CH, TW), jnp.float32),
            pltpu.VMEM((SC_CH // SC_JB, SC_JB), jnp.int32),
        ],
        compiler_params=pltpu.CompilerParams(
            needs_layout_passes=False, use_tc_tiling_on_sc=False
        ),
    )
    return k(tab, pos3)


def _bf_body(ge_smem, vs, gs, ds, vcol, gcol, dcol, lab, drow, rc_row,
             cdd_out, pcc_out, rs_acc):
    b = pl.program_id(0)
    base = b * BI2
    gi = gs[:, pl.ds(base, BI2)]
    vi = vs[:, pl.ds(base, BI2)]
    di = ds[:, pl.ds(base, BI2)]
    gmax = jnp.max(gi).astype(jnp.int32)
    jend = ge_smem[gmax]
    nch = (jend - base + CJB - 1) // CJB
    dik = di + K_MARGIN

    def body(c, s):
        st = base + c * CJB
        vj = vcol[pl.ds(st, CJB), :]
        gj = gcol[pl.ds(st, CJB), :]
        dj = dcol[pl.ds(st, CJB), :]
        mrank = (gj == gi) & (vj > vi)
        val = jnp.maximum(dik - dj, 0.0)
        return s + jnp.sum(jnp.where(mrank, val, 0.0), axis=0, keepdims=True)

    s = lax.fori_loop(0, nch, body, jnp.zeros((1, BI2), jnp.float32))

    iota_g = lax.broadcasted_iota(jnp.int32, (1, G), 1).astype(jnp.float32)
    onehot_blk = (gcol[pl.ds(base, BI2), :] == iota_g).astype(jnp.float32)
    contrib = lax.dot_general(
        s, onehot_blk, (((1,), (0,)), ((), ())),
        preferred_element_type=jnp.float32,
        precision=lax.Precision.HIGHEST,
    )

    @pl.when(b == 0)
    def _():
        rs_acc[...] = contrib

    @pl.when(b > 0)
    def _():
        rs_acc[...] += contrib

    @pl.when(b == NB2 - 1)
    def _():
        labf = lab[...]
        d = drow[...]
        mb = (labf == 0).astype(jnp.float32)
        mp = (labf == 1).astype(jnp.float32)
        cb = jnp.sum(mb)
        cp = jnp.sum(mp)
        mean_b = jnp.sum(mb * d) / jnp.maximum(cb, 1.0)
        mean_p = jnp.sum(mp * d) / jnp.maximum(cp, 1.0)
        l_cdd = jnp.where(
            (cb > 0) & (cp > 0), jnp.maximum(MARGIN + mean_b - mean_p, 0.0), 0.0
        )
        cdd_out[...] = jnp.reshape(l_cdd, (1, 1))

        v = vs[:, :N]
        g = gs[:, :N]
        dsv = ds[:, :N]
        pad1 = jnp.full((1, 1), -1.0, jnp.float32)
        g_next = jnp.concatenate([g[:, 1:], pad1], axis=1)
        d_next = jnp.concatenate([dsv[:, 1:], pad1], axis=1)
        neigh = jnp.where(
            g_next == g, jnp.maximum(dsv - d_next + K_MARGIN, 0.0), 0.0
        )

        ones = jnp.ones((1, N), jnp.float32)
        feats = jnp.concatenate(
            [ones, v, dsv, v * v, dsv * dsv, v * dsv, neigh], axis=0
        )
        onehot = (
            gcol[pl.ds(0, N), :]
            == lax.broadcasted_iota(jnp.int32, (1, G), 1).astype(jnp.float32)
        ).astype(jnp.float32)
        seg = lax.dot_general(
            feats, onehot, (((1,), (0,)), ((), ())),
            preferred_element_type=jnp.float32,
            precision=lax.Precision.HIGHEST,
        )

        n = seg[0:1, :]
        sv = seg[1:2, :]
        sd = seg[2:3, :]
        sv2 = seg[3:4, :]
        sd2 = seg[4:5, :]
        svd = seg[5:6, :]
        nv = seg[6:7, :]
        rs = rs_acc[...]
        rc = rc_row[...]

        dn = jnp.maximum(n, 1.0)
        dn1 = jnp.maximum(n - 1.0, 1.0)
        mv = sv / dn
        md = sd / dn
        ssv = jnp.maximum(sv2 - n * mv * mv, 0.0)
        ssd = jnp.maximum(sd2 - n * md * md, 0.0)
        cvd = svd - n * mv * md
        stdv = jnp.sqrt(ssv / dn1)
        stdd = jnp.sqrt(ssd / dn1)
        e = 1e-6
        iv = 1.0 / (stdv + e)
        id_ = 1.0 / (stdd + e)
        corr_mean = (ssv * iv * iv - 2.0 * cvd * iv * id_ + ssd * id_ * id_) / dn
        corr_loss = jnp.where((stdv > 0) & (stdd > 0), corr_mean, 0.0)

        neigh_viol = nv / dn1
        rank_loss = jnp.where(rc > 0, rs / jnp.maximum(rc, 1.0), 0.0)

        group_loss = corr_loss + neigh_viol + rank_loss
        valid = n >= 2.0
        sizes = jnp.where(valid, n, 0.0)
        total = jnp.sum(sizes)
        weights = sizes / jnp.maximum(total, 1.0)
        l_pcc = jnp.where(
            total > 0, jnp.sum(jnp.where(valid, weights * group_loss, 0.0)), 0.0
        )
        pcc_out[...] = jnp.reshape(l_pcc, (1, 1))


def _bf_call(ge, vs_row, gs_row, ds_row, vs_colp, gs_colp, ds_colp, lab_row,
             d_row, rc_row):
    rowp = lambda: pl.BlockSpec((1, NP), lambda b: (0, 0))
    col = lambda: pl.BlockSpec((NP, 1), lambda b: (0, 0))
    return pl.pallas_call(
        _bf_body,
        grid=(NB2,),
        in_specs=[
            pl.BlockSpec(memory_space=pltpu.SMEM),
            rowp(), rowp(), rowp(), col(), col(), col(),
            pl.BlockSpec((1, N), lambda b: (0, 0)),
            pl.BlockSpec((1, N), lambda b: (0, 0)),
            pl.BlockSpec((1, G), lambda b: (0, 0)),
        ],
        out_specs=[pl.BlockSpec((1, 1), lambda b: (0, 0)),
                   pl.BlockSpec((1, 1), lambda b: (0, 0))],
        out_shape=[
            jax.ShapeDtypeStruct((1, 1), jnp.float32),
            jax.ShapeDtypeStruct((1, 1), jnp.float32),
        ],
        scratch_shapes=[pltpu.VMEM((1, G), jnp.float32)],
    )(ge, vs_row, gs_row, ds_row, vs_colp, gs_colp, ds_colp, lab_row, d_row,
      rc_row)


def kernel(z_r, z_v, labels, groups, var_lens):
    labels = labels.astype(jnp.int32)
    groups = groups.astype(jnp.int32)
    var_lens = var_lens.astype(jnp.int32)

    v_col = var_lens.reshape(N, 1)
    g_col = groups.reshape(N, 1)
    v_row = var_lens.reshape(1, N)
    g_row = groups.reshape(1, N)

    d_col, tab = _dot_call(z_r, z_v, v_col, g_col)
    pos_col, ge_col, rc_row = _pos_call(g_col, v_col, g_row, v_row)

    tab_s = _sc_scatter(tab, pos_col.reshape(SC_NW, SC_CH // SC_JB, SC_JB))

    vs_col = tab_s[:, 0:1]
    gs_colf = tab_s[:, 1:2]
    ds_col = tab_s[:, 2:3]
    padv = jnp.zeros((CJB, 1), jnp.float32)
    padg = jnp.full((CJB, 1), -2.0, jnp.float32)
    vs_colp = jnp.concatenate([vs_col, padv], axis=0)
    gs_colp = jnp.concatenate([gs_colf, padg], axis=0)
    ds_colp = jnp.concatenate([ds_col, padv], axis=0)
    vs_rowp = vs_colp.reshape(1, NP)
    gs_rowp = gs_colp.reshape(1, NP)
    ds_rowp = ds_colp.reshape(1, NP)

    cdd, pcc = _bf_call(
        ge_col.reshape(G),
        vs_rowp,
        gs_rowp,
        ds_rowp,
        vs_colp,
        gs_colp,
        ds_colp,
        labels.reshape(1, N),
        d_col.reshape(1, N),
        rc_row,
    )
    return cdd[0, 0], pcc[0, 0], d_col.reshape(N)


# speedup vs baseline: 1.0123x; 1.0123x over previous
import jax
import jax.numpy as jnp
from jax import lax
from jax.experimental import pallas as pl
from jax.experimental.pallas import tpu as pltpu
from jax.experimental.pallas import tpu_sc as plsc

MARGIN = 2.0
K_MARGIN = 0.02
N = 16384
D_FEAT = 64
G = 256
TW = 16

C_POS = 512
NC_POS = N // C_POS

BI2 = 256
CJB = 512
NB2 = N // BI2
NP = N + CJB

SC_NC = 2
SC_NS = 16
SC_NW = SC_NC * SC_NS
SC_CH = N // SC_NW
SC_JB = 128


def _dot_body(zr_ref, zv_ref, v_ref, g_ref, d_ref, tab_ref):
    d = 1.0 - jnp.sum(zr_ref[...] * zv_ref[...], axis=1, keepdims=True)
    d_ref[...] = d
    vf = v_ref[...].astype(jnp.float32)
    gf = g_ref[...].astype(jnp.float32)
    blk = d.shape[0]
    pad = jnp.zeros((blk, TW - 3), jnp.float32)
    tab_ref[...] = jnp.concatenate([vf, gf, d, pad], axis=1)


def _dot_call(z_r, z_v, v_col, g_col):
    blk = 1024
    return pl.pallas_call(
        _dot_body,
        grid=(N // blk,),
        in_specs=[
            pl.BlockSpec((blk, D_FEAT), lambda b: (b, 0)),
            pl.BlockSpec((blk, D_FEAT), lambda b: (b, 0)),
            pl.BlockSpec((blk, 1), lambda b: (b, 0)),
            pl.BlockSpec((blk, 1), lambda b: (b, 0)),
        ],
        out_specs=[
            pl.BlockSpec((blk, 1), lambda b: (b, 0)),
            pl.BlockSpec((blk, TW), lambda b: (b, 0)),
        ],
        out_shape=[
            jax.ShapeDtypeStruct((N, 1), jnp.float32),
            jax.ShapeDtypeStruct((N, TW), jnp.float32),
        ],
    )(z_r, z_v, v_col, g_col)


def _mm(x, y):
    return lax.dot_general(
        x, y, (((1,), (0,)), ((), ())), preferred_element_type=jnp.float32
    )


def _mm_hi(x, y):
    return lax.dot_general(
        x, y, (((1,), (0,)), ((), ())),
        preferred_element_type=jnp.float32,
        precision=lax.Precision.HIGHEST,
    )


def _pos_body(gcol, vcol, grow, vrow, pos_out, ge_out, rc_out, htT, hcT, hist):
    iota_row = lax.broadcasted_iota(jnp.int32, (1, G), 1)
    iota_col = lax.broadcasted_iota(jnp.int32, (G, 1), 0)

    def mats(c):
        gc = gcol[pl.ds(c * C_POS, C_POS), :]
        vc = vcol[pl.ds(c * C_POS, C_POS), :]
        gr = grow[:, pl.ds(c * C_POS, C_POS)]
        vr = vrow[:, pl.ds(c * C_POS, C_POS)]
        a = (gc == iota_row).astype(jnp.float32)
        b = (vc == iota_row).astype(jnp.float32)
        bT = (iota_col == vr).astype(jnp.float32)
        return gc, vc, gr, vr, a, b, bT

    htT[...] = jnp.zeros((G, G), jnp.float32)

    def l1(c, _):
        _, _, _, _, a, _, bT = mats(c)
        h = _mm(bT, a)
        hist[pl.ds(c * G, G), :] = h
        htT[...] += h
        return 0

    lax.fori_loop(0, NC_POS, l1, 0)

    ht = htT[...]
    slv = (iota_col > iota_row).astype(jnp.float32)
    sug = (iota_col < iota_row).astype(jnp.float32)
    rowcumT = _mm_hi(slv, ht)
    t_row = jnp.sum(ht, axis=0, keepdims=True)
    texT = _mm_hi(t_row, sug)
    offT = texT + rowcumT
    ge_out[...] = (texT + t_row).astype(jnp.int32)
    sumsq = jnp.sum(ht * ht, axis=0, keepdims=True)
    rc_out[...] = (t_row * t_row - sumsq) * 0.5

    hcT[...] = jnp.zeros((G, G), jnp.float32)
    iota_i = lax.broadcasted_iota(jnp.int32, (C_POS, 1), 0)
    iota_j = lax.broadcasted_iota(jnp.int32, (1, C_POS), 1)

    def l2(c, _):
        gc, vc, gr, vr, a, b, bT = mats(c)
        tcT = offT + hcT[...]
        tc_hi = jnp.floor(tcT * (1.0 / 128.0))
        tc_lo = tcT - tc_hi * 128.0
        m = _mm(b, tc_hi) * 128.0 + _mm(b, tc_lo)
        lookup = jnp.sum(a * m, axis=1, keepdims=True)
        meq = (gc == gr) & (vc == vr) & (iota_j < iota_i)
        r = jnp.sum(meq.astype(jnp.float32), axis=1, keepdims=True)
        pos_out[pl.ds(c * C_POS, C_POS), :] = (lookup + r).astype(jnp.int32)
        hcT[...] += hist[pl.ds(c * G, G), :]
        return 0

    lax.fori_loop(0, NC_POS, l2, 0)


def _pos_call(g_col, v_col, g_row, v_row):
    return pl.pallas_call(
        _pos_body,
        in_specs=[
            pl.BlockSpec((N, 1), lambda: (0, 0)),
            pl.BlockSpec((N, 1), lambda: (0, 0)),
            pl.BlockSpec((1, N), lambda: (0, 0)),
            pl.BlockSpec((1, N), lambda: (0, 0)),
        ],
        out_specs=[
            pl.BlockSpec((N, 1), lambda: (0, 0)),
            pl.BlockSpec((1, G), lambda: (0, 0)),
            pl.BlockSpec((1, G), lambda: (0, 0)),
        ],
        out_shape=[
            jax.ShapeDtypeStruct((N, 1), jnp.int32),
            jax.ShapeDtypeStruct((1, G), jnp.int32),
            jax.ShapeDtypeStruct((1, G), jnp.float32),
        ],
        scratch_shapes=[
            pltpu.VMEM((G, G), jnp.float32),
            pltpu.VMEM((G, G), jnp.float32),
            pltpu.VMEM((NC_POS * G, G), jnp.float32),
        ],
    )(g_col, v_col, g_row, v_row)


def _sc_scatter_body(tab_hbm, pos_hbm, out_hbm, tab_v, pos_v):
    wid = lax.axis_index("s") * SC_NC + lax.axis_index("c")
    base = wid * SC_CH
    pltpu.sync_copy(tab_hbm.at[pl.ds(base, SC_CH)], tab_v)
    pltpu.sync_copy(pos_hbm.at[wid], pos_v)
    for j in range(SC_CH // SC_JB):
        pltpu.sync_copy(
            tab_v.at[pl.ds(j * SC_JB, SC_JB)], out_hbm.at[pos_v.at[j]]
        )


def _sc_scatter(tab, pos3):
    k = pl.kernel(
        _sc_scatter_body,
        out_type=jax.ShapeDtypeStruct((N, TW), jnp.float32),
        mesh=plsc.VectorSubcoreMesh(core_axis_name="c", subcore_axis_name="s"),
        scratch_types=[
            pltpu.VMEM((SC_CH, TW), jnp.float32),
            pltpu.VMEM((SC_CH // SC_JB, SC_JB), jnp.int32),
        ],
        compiler_params=pltpu.CompilerParams(
            needs_layout_passes=False, use_tc_tiling_on_sc=False
        ),
    )
    return k(tab, pos3)


def _pair2_body(ge_smem, vcol, gcol, dcol, vrow, grow, drow, sum_out):
    b = pl.program_id(0)
    base = b * BI2
    gi = gcol[...]
    vi = vcol[...]
    di = dcol[...]
    gmax = jnp.max(gi).astype(jnp.int32)
    jend = ge_smem[gmax]
    nch = (jend - base + CJB - 1) // CJB
    dik = di + K_MARGIN

    def body(c, s):
        st = base + c * CJB
        vj = vrow[:, pl.ds(st, CJB)]
        gj = grow[:, pl.ds(st, CJB)]
        dj = drow[:, pl.ds(st, CJB)]
        mrank = (gj == gi) & (vj > vi)
        val = jnp.maximum(dik - dj, 0.0)
        return s + jnp.sum(jnp.where(mrank, val, 0.0), axis=1, keepdims=True)

    s = lax.fori_loop(0, nch, body, jnp.zeros((BI2, 1), jnp.float32))
    sum_out[...] = s


def _pair2_call(ge, vs_col, gs_col, ds_col, vs_rowp, gs_rowp, ds_rowp):
    col = lambda: pl.BlockSpec((BI2, 1), lambda b: (b, 0))
    row = lambda: pl.BlockSpec((1, NP), lambda b: (0, 0))
    return pl.pallas_call(
        _pair2_body,
        grid=(NB2,),
        in_specs=[
            pl.BlockSpec(memory_space=pltpu.SMEM),
            col(), col(), col(), row(), row(), row(),
        ],
        out_specs=col(),
        out_shape=jax.ShapeDtypeStruct((N, 1), jnp.float32),
    )(ge, vs_col, gs_col, ds_col, vs_rowp, gs_rowp, ds_rowp)


def _final_body(lab, drow, vs, gs, ds, rsum, rc_row, gs_col, cdd_out, pcc_out):
    labf = lab[...]
    d = drow[...]

    mb = (labf == 0).astype(jnp.float32)
    mp = (labf == 1).astype(jnp.float32)
    cb = jnp.sum(mb)
    cp = jnp.sum(mp)
    mean_b = jnp.sum(mb * d) / jnp.maximum(cb, 1.0)
    mean_p = jnp.sum(mp * d) / jnp.maximum(cp, 1.0)
    l_cdd = jnp.where(
        (cb > 0) & (cp > 0), jnp.maximum(MARGIN + mean_b - mean_p, 0.0), 0.0
    )
    cdd_out[...] = jnp.reshape(l_cdd, (1, 1))

    v = vs[...]
    g = gs[...]
    dsv = ds[...]
    pad1 = jnp.full((1, 1), -1.0, jnp.float32)
    g_next = jnp.concatenate([g[:, 1:], pad1], axis=1)
    d_next = jnp.concatenate([dsv[:, 1:], pad1], axis=1)
    neigh = jnp.where(
        g_next == g, jnp.maximum(dsv - d_next + K_MARGIN, 0.0), 0.0
    )

    ones = jnp.ones((1, N), jnp.float32)
    feats = jnp.concatenate(
        [ones, v, dsv, v * v, dsv * dsv, v * dsv, rsum[...], neigh],
        axis=0,
    )
    onehot = (gs_col[...] == lax.broadcasted_iota(jnp.int32, (1, G), 1)).astype(
        jnp.float32
    )
    seg = lax.dot_general(
        feats, onehot, (((1,), (0,)), ((), ())),
        preferred_element_type=jnp.float32,
        precision=lax.Precision.HIGHEST,
    )

    n = seg[0:1, :]
    sv = seg[1:2, :]
    sd = seg[2:3, :]
    sv2 = seg[3:4, :]
    sd2 = seg[4:5, :]
    svd = seg[5:6, :]
    rs = seg[6:7, :]
    nv = seg[7:8, :]
    rc = rc_row[...]

    dn = jnp.maximum(n, 1.0)
    dn1 = jnp.maximum(n - 1.0, 1.0)
    mv = sv / dn
    md = sd / dn
    ssv = jnp.maximum(sv2 - n * mv * mv, 0.0)
    ssd = jnp.maximum(sd2 - n * md * md, 0.0)
    cvd = svd - n * mv * md
    stdv = jnp.sqrt(ssv / dn1)
    stdd = jnp.sqrt(ssd / dn1)
    e = 1e-6
    iv = 1.0 / (stdv + e)
    id_ = 1.0 / (stdd + e)
    corr_mean = (ssv * iv * iv - 2.0 * cvd * iv * id_ + ssd * id_ * id_) / dn
    corr_loss = jnp.where((stdv > 0) & (stdd > 0), corr_mean, 0.0)

    neigh_viol = nv / dn1
    rank_loss = jnp.where(rc > 0, rs / jnp.maximum(rc, 1.0), 0.0)

    group_loss = corr_loss + neigh_viol + rank_loss
    valid = n >= 2.0
    sizes = jnp.where(valid, n, 0.0)
    total = jnp.sum(sizes)
    weights = sizes / jnp.maximum(total, 1.0)
    l_pcc = jnp.where(
        total > 0, jnp.sum(jnp.where(valid, weights * group_loss, 0.0)), 0.0
    )
    pcc_out[...] = jnp.reshape(l_pcc, (1, 1))


def _final_call(lab_row, d_row, vs_row, gs_row, ds_row, rsum_row, rc_row, gs_col):
    row_i = pl.BlockSpec((1, N), lambda: (0, 0))
    return pl.pallas_call(
        _final_body,
        in_specs=[row_i, row_i, row_i, row_i, row_i, row_i,
                  pl.BlockSpec((1, G), lambda: (0, 0)),
                  pl.BlockSpec((N, 1), lambda: (0, 0))],
        out_specs=[pl.BlockSpec((1, 1), lambda: (0, 0)),
                   pl.BlockSpec((1, 1), lambda: (0, 0))],
        out_shape=[
            jax.ShapeDtypeStruct((1, 1), jnp.float32),
            jax.ShapeDtypeStruct((1, 1), jnp.float32),
        ],
    )(lab_row, d_row, vs_row, gs_row, ds_row, rsum_row, rc_row, gs_col)


def kernel(z_r, z_v, labels, groups, var_lens):
    labels = labels.astype(jnp.int32)
    groups = groups.astype(jnp.int32)
    var_lens = var_lens.astype(jnp.int32)

    v_col = var_lens.reshape(N, 1)
    g_col = groups.reshape(N, 1)
    v_row = var_lens.reshape(1, N)
    g_row = groups.reshape(1, N)

    d_col, tab = _dot_call(z_r, z_v, v_col, g_col)
    pos_col, ge_col, rc_row = _pos_call(g_col, v_col, g_row, v_row)

    tab_s = _sc_scatter(tab, pos_col.reshape(SC_NW, SC_CH // SC_JB, SC_JB))

    vs_col = tab_s[:, 0:1]
    gs_colf = tab_s[:, 1:2]
    ds_col = tab_s[:, 2:3]
    vs_row = vs_col.reshape(1, N)
    gs_row = gs_colf.reshape(1, N)
    ds_row = ds_col.reshape(1, N)
    padv = jnp.zeros((1, CJB), jnp.float32)
    padg = jnp.full((1, CJB), -2.0, jnp.float32)
    vs_rowp = jnp.concatenate([vs_row, padv], axis=1)
    gs_rowp = jnp.concatenate([gs_row, padg], axis=1)
    ds_rowp = jnp.concatenate([ds_row, padv], axis=1)

    rsum_col = _pair2_call(
        ge_col.reshape(G), vs_col, gs_colf, ds_col, vs_rowp, gs_rowp, ds_rowp
    )
    rsum_row = rsum_col.reshape(1, N)

    cdd, pcc = _final_call(
        labels.reshape(1, N),
        d_col.reshape(1, N),
        vs_row,
        gs_row,
        ds_row,
        rsum_row,
        rc_row,
        gs_colf.astype(jnp.int32),
    )
    return cdd[0, 0], pcc[0, 0], d_col.reshape(N)


# speedup vs baseline: 1.0679x; 1.0549x over previous
import jax
import jax.numpy as jnp
from jax import lax
from jax.experimental import pallas as pl
from jax.experimental.pallas import tpu as pltpu
from jax.experimental.pallas import tpu_sc as plsc

MARGIN = 2.0
K_MARGIN = 0.02
N = 16384
D_FEAT = 64
G = 256
TW = 16

C_POS = 512
NC_POS = N // C_POS

BI2 = 512
CJB = 256
NB2 = N // BI2
NP = N + CJB

SC_NC = 2
SC_NS = 16
SC_NW = SC_NC * SC_NS
SC_CH = N // SC_NW
SC_JB = 128


def _dot_body(zr_ref, zv_ref, v_ref, g_ref, d_ref, tab_ref):
    d = 1.0 - jnp.sum(zr_ref[...] * zv_ref[...], axis=1, keepdims=True)
    d_ref[...] = d
    vf = v_ref[...].astype(jnp.float32)
    gf = g_ref[...].astype(jnp.float32)
    blk = d.shape[0]
    pad = jnp.zeros((blk, TW - 3), jnp.float32)
    tab_ref[...] = jnp.concatenate([vf, gf, d, pad], axis=1)


def _dot_call(z_r, z_v, v_col, g_col):
    blk = 1024
    return pl.pallas_call(
        _dot_body,
        grid=(N // blk,),
        in_specs=[
            pl.BlockSpec((blk, D_FEAT), lambda b: (b, 0)),
            pl.BlockSpec((blk, D_FEAT), lambda b: (b, 0)),
            pl.BlockSpec((blk, 1), lambda b: (b, 0)),
            pl.BlockSpec((blk, 1), lambda b: (b, 0)),
        ],
        out_specs=[
            pl.BlockSpec((blk, 1), lambda b: (b, 0)),
            pl.BlockSpec((blk, TW), lambda b: (b, 0)),
        ],
        out_shape=[
            jax.ShapeDtypeStruct((N, 1), jnp.float32),
            jax.ShapeDtypeStruct((N, TW), jnp.float32),
        ],
    )(z_r, z_v, v_col, g_col)


def _mm(x, y):
    return lax.dot_general(
        x, y, (((1,), (0,)), ((), ())), preferred_element_type=jnp.float32
    )


def _mm_hi(x, y):
    return lax.dot_general(
        x, y, (((1,), (0,)), ((), ())),
        preferred_element_type=jnp.float32,
        precision=lax.Precision.HIGHEST,
    )


def _pos_body(gcol, vcol, grow, vrow, pos_out, ge_out, rc_out, htT, hcT, hist):
    iota_row = lax.broadcasted_iota(jnp.int32, (1, G), 1)
    iota_col = lax.broadcasted_iota(jnp.int32, (G, 1), 0)

    def mats(c):
        gc = gcol[pl.ds(c * C_POS, C_POS), :]
        vc = vcol[pl.ds(c * C_POS, C_POS), :]
        gr = grow[:, pl.ds(c * C_POS, C_POS)]
        vr = vrow[:, pl.ds(c * C_POS, C_POS)]
        a = (gc == iota_row).astype(jnp.float32)
        b = (vc == iota_row).astype(jnp.float32)
        bT = (iota_col == vr).astype(jnp.float32)
        return gc, vc, gr, vr, a, b, bT

    htT[...] = jnp.zeros((G, G), jnp.float32)

    def l1(c, _):
        _, _, _, _, a, _, bT = mats(c)
        h = _mm(bT, a)
        hist[pl.ds(c * G, G), :] = h
        htT[...] += h
        return 0

    lax.fori_loop(0, NC_POS, l1, 0)

    ht = htT[...]
    slv = (iota_col > iota_row).astype(jnp.float32)
    sug = (iota_col < iota_row).astype(jnp.float32)
    rowcumT = _mm_hi(slv, ht)
    t_row = jnp.sum(ht, axis=0, keepdims=True)
    texT = _mm_hi(t_row, sug)
    offT = texT + rowcumT
    ge_out[...] = (texT + t_row).astype(jnp.int32)
    sumsq = jnp.sum(ht * ht, axis=0, keepdims=True)
    rc_out[...] = (t_row * t_row - sumsq) * 0.5

    hcT[...] = jnp.zeros((G, G), jnp.float32)
    iota_i = lax.broadcasted_iota(jnp.int32, (C_POS, 1), 0)
    iota_j = lax.broadcasted_iota(jnp.int32, (1, C_POS), 1)

    def l2(c, _):
        gc, vc, gr, vr, a, b, bT = mats(c)
        tcT = offT + hcT[...]
        tc_hi = jnp.floor(tcT * (1.0 / 128.0))
        tc_lo = tcT - tc_hi * 128.0
        m = _mm(b, tc_hi) * 128.0 + _mm(b, tc_lo)
        lookup = jnp.sum(a * m, axis=1, keepdims=True)
        meq = (gc == gr) & (vc == vr) & (iota_j < iota_i)
        r = jnp.sum(meq.astype(jnp.float32), axis=1, keepdims=True)
        pos_out[pl.ds(c * C_POS, C_POS), :] = (lookup + r).astype(jnp.int32)
        hcT[...] += hist[pl.ds(c * G, G), :]
        return 0

    lax.fori_loop(0, NC_POS, l2, 0)


def _pos_call(g_col, v_col, g_row, v_row):
    return pl.pallas_call(
        _pos_body,
        in_specs=[
            pl.BlockSpec((N, 1), lambda: (0, 0)),
            pl.BlockSpec((N, 1), lambda: (0, 0)),
            pl.BlockSpec((1, N), lambda: (0, 0)),
            pl.BlockSpec((1, N), lambda: (0, 0)),
        ],
        out_specs=[
            pl.BlockSpec((N, 1), lambda: (0, 0)),
            pl.BlockSpec((1, G), lambda: (0, 0)),
            pl.BlockSpec((1, G), lambda: (0, 0)),
        ],
        out_shape=[
            jax.ShapeDtypeStruct((N, 1), jnp.int32),
            jax.ShapeDtypeStruct((1, G), jnp.int32),
            jax.ShapeDtypeStruct((1, G), jnp.float32),
        ],
        scratch_shapes=[
            pltpu.VMEM((G, G), jnp.float32),
            pltpu.VMEM((G, G), jnp.float32),
            pltpu.VMEM((NC_POS * G, G), jnp.float32),
        ],
    )(g_col, v_col, g_row, v_row)


def _sc_scatter_body(tab_hbm, pos_hbm, out_hbm, tab_v, pos_v):
    wid = lax.axis_index("s") * SC_NC + lax.axis_index("c")
    base = wid * SC_CH
    pltpu.sync_copy(tab_hbm.at[pl.ds(base, SC_CH)], tab_v)
    pltpu.sync_copy(pos_hbm.at[wid], pos_v)
    for j in range(SC_CH // SC_JB):
        pltpu.sync_copy(
            tab_v.at[pl.ds(j * SC_JB, SC_JB)], out_hbm.at[pos_v.at[j]]
        )


def _sc_scatter(tab, pos3):
    k = pl.kernel(
        _sc_scatter_body,
        out_type=jax.ShapeDtypeStruct((N, TW), jnp.float32),
        mesh=plsc.VectorSubcoreMesh(core_axis_name="c", subcore_axis_name="s"),
        scratch_types=[
            pltpu.VMEM((SC_CH, TW), jnp.float32),
            pltpu.VMEM((SC_CH // SC_JB, SC_JB), jnp.int32),
        ],
        compiler_params=pltpu.CompilerParams(
            needs_layout_passes=False, use_tc_tiling_on_sc=False
        ),
    )
    return k(tab, pos3)


def _pair2_body(ge_smem, vrow, grow, drow, vcol, gcol, dcol, sum_out):
    b = pl.program_id(0)
    base = b * BI2
    gi = grow[...]
    vi = vrow[...]
    di = drow[...]
    gmax = jnp.max(gi).astype(jnp.int32)
    jend = ge_smem[gmax]
    nch = (jend - base + CJB - 1) // CJB
    dik = di + K_MARGIN

    def body(c, s):
        st = base + c * CJB
        vj = vcol[pl.ds(st, CJB), :]
        gj = gcol[pl.ds(st, CJB), :]
        dj = dcol[pl.ds(st, CJB), :]
        mrank = (gj == gi) & (vj > vi)
        val = jnp.maximum(dik - dj, 0.0)
        return s + jnp.sum(jnp.where(mrank, val, 0.0), axis=0, keepdims=True)

    s = lax.fori_loop(0, nch, body, jnp.zeros((1, BI2), jnp.float32))
    sum_out[...] = s


def _pair2_call(ge, vs_row, gs_row, ds_row, vs_colp, gs_colp, ds_colp):
    row = lambda: pl.BlockSpec((1, BI2), lambda b: (0, b))
    col = lambda: pl.BlockSpec((NP, 1), lambda b: (0, 0))
    return pl.pallas_call(
        _pair2_body,
        grid=(NB2,),
        in_specs=[
            pl.BlockSpec(memory_space=pltpu.SMEM),
            row(), row(), row(), col(), col(), col(),
        ],
        out_specs=row(),
        out_shape=jax.ShapeDtypeStruct((1, N), jnp.float32),
    )(ge, vs_row, gs_row, ds_row, vs_colp, gs_colp, ds_colp)


def _final_body(lab, drow, vs, gs, ds, rsum, rc_row, gs_col, cdd_out, pcc_out):
    labf = lab[...]
    d = drow[...]

    mb = (labf == 0).astype(jnp.float32)
    mp = (labf == 1).astype(jnp.float32)
    cb = jnp.sum(mb)
    cp = jnp.sum(mp)
    mean_b = jnp.sum(mb * d) / jnp.maximum(cb, 1.0)
    mean_p = jnp.sum(mp * d) / jnp.maximum(cp, 1.0)
    l_cdd = jnp.where(
        (cb > 0) & (cp > 0), jnp.maximum(MARGIN + mean_b - mean_p, 0.0), 0.0
    )
    cdd_out[...] = jnp.reshape(l_cdd, (1, 1))

    v = vs[...]
    g = gs[...]
    dsv = ds[...]
    pad1 = jnp.full((1, 1), -1.0, jnp.float32)
    g_next = jnp.concatenate([g[:, 1:], pad1], axis=1)
    d_next = jnp.concatenate([dsv[:, 1:], pad1], axis=1)
    neigh = jnp.where(
        g_next == g, jnp.maximum(dsv - d_next + K_MARGIN, 0.0), 0.0
    )

    ones = jnp.ones((1, N), jnp.float32)
    feats = jnp.concatenate(
        [ones, v, dsv, v * v, dsv * dsv, v * dsv, rsum[...], neigh],
        axis=0,
    )
    onehot = (gs_col[...] == lax.broadcasted_iota(jnp.int32, (1, G), 1)).astype(
        jnp.float32
    )
    seg = lax.dot_general(
        feats, onehot, (((1,), (0,)), ((), ())),
        preferred_element_type=jnp.float32,
        precision=lax.Precision.HIGHEST,
    )

    n = seg[0:1, :]
    sv = seg[1:2, :]
    sd = seg[2:3, :]
    sv2 = seg[3:4, :]
    sd2 = seg[4:5, :]
    svd = seg[5:6, :]
    rs = seg[6:7, :]
    nv = seg[7:8, :]
    rc = rc_row[...]

    dn = jnp.maximum(n, 1.0)
    dn1 = jnp.maximum(n - 1.0, 1.0)
    mv = sv / dn
    md = sd / dn
    ssv = jnp.maximum(sv2 - n * mv * mv, 0.0)
    ssd = jnp.maximum(sd2 - n * md * md, 0.0)
    cvd = svd - n * mv * md
    stdv = jnp.sqrt(ssv / dn1)
    stdd = jnp.sqrt(ssd / dn1)
    e = 1e-6
    iv = 1.0 / (stdv + e)
    id_ = 1.0 / (stdd + e)
    corr_mean = (ssv * iv * iv - 2.0 * cvd * iv * id_ + ssd * id_ * id_) / dn
    corr_loss = jnp.where((stdv > 0) & (stdd > 0), corr_mean, 0.0)

    neigh_viol = nv / dn1
    rank_loss = jnp.where(rc > 0, rs / jnp.maximum(rc, 1.0), 0.0)

    group_loss = corr_loss + neigh_viol + rank_loss
    valid = n >= 2.0
    sizes = jnp.where(valid, n, 0.0)
    total = jnp.sum(sizes)
    weights = sizes / jnp.maximum(total, 1.0)
    l_pcc = jnp.where(
        total > 0, jnp.sum(jnp.where(valid, weights * group_loss, 0.0)), 0.0
    )
    pcc_out[...] = jnp.reshape(l_pcc, (1, 1))


def _final_call(lab_row, d_row, vs_row, gs_row, ds_row, rsum_row, rc_row, gs_col):
    row_i = pl.BlockSpec((1, N), lambda: (0, 0))
    return pl.pallas_call(
        _final_body,
        in_specs=[row_i, row_i, row_i, row_i, row_i, row_i,
                  pl.BlockSpec((1, G), lambda: (0, 0)),
                  pl.BlockSpec((N, 1), lambda: (0, 0))],
        out_specs=[pl.BlockSpec((1, 1), lambda: (0, 0)),
                   pl.BlockSpec((1, 1), lambda: (0, 0))],
        out_shape=[
            jax.ShapeDtypeStruct((1, 1), jnp.float32),
            jax.ShapeDtypeStruct((1, 1), jnp.float32),
        ],
    )(lab_row, d_row, vs_row, gs_row, ds_row, rsum_row, rc_row, gs_col)


def kernel(z_r, z_v, labels, groups, var_lens):
    labels = labels.astype(jnp.int32)
    groups = groups.astype(jnp.int32)
    var_lens = var_lens.astype(jnp.int32)

    v_col = var_lens.reshape(N, 1)
    g_col = groups.reshape(N, 1)
    v_row = var_lens.reshape(1, N)
    g_row = groups.reshape(1, N)

    d_col, tab = _dot_call(z_r, z_v, v_col, g_col)
    pos_col, ge_col, rc_row = _pos_call(g_col, v_col, g_row, v_row)

    tab_s = _sc_scatter(tab, pos_col.reshape(SC_NW, SC_CH // SC_JB, SC_JB))

    vs_col = tab_s[:, 0:1]
    gs_colf = tab_s[:, 1:2]
    ds_col = tab_s[:, 2:3]
    vs_row = vs_col.reshape(1, N)
    gs_row = gs_colf.reshape(1, N)
    ds_row = ds_col.reshape(1, N)
    padv = jnp.zeros((CJB, 1), jnp.float32)
    padg = jnp.full((CJB, 1), -2.0, jnp.float32)
    vs_colp = jnp.concatenate([vs_col, padv], axis=0)
    gs_colp = jnp.concatenate([gs_colf, padg], axis=0)
    ds_colp = jnp.concatenate([ds_col, padv], axis=0)

    rsum_row = _pair2_call(
        ge_col.reshape(G), vs_row, gs_row, ds_row, vs_colp, gs_colp, ds_colp
    )

    cdd, pcc = _final_call(
        labels.reshape(1, N),
        d_col.reshape(1, N),
        vs_row,
        gs_row,
        ds_row,
        rsum_row,
        rc_row,
        gs_colf.astype(jnp.int32),
    )
    return cdd[0, 0], pcc[0, 0], d_col.reshape(N)
